# baseline (speedup 1.0000x reference)
import functools

import jax
import jax.numpy as jnp
from jax import lax
from jax.experimental import pallas as pl
from jax.experimental.pallas import tpu as pltpu
from jax.experimental.pallas import tpu_sc as plsc

LAT_BINS = 100000
LON_BINS = 100000
EMBED_DIM = 64
BATCH = 16384

_info = plsc.get_sparse_core_info()
_NC = _info.num_cores
_NS = _info.num_subcores
_NW = _NC * _NS
_BPW = BATCH // _NW
_CH = 128
_NCHUNK = _BPW // _CH


def _body(lat_idx_hbm, lon_idx_hbm, lat_t_hbm, lon_t_hbm, out_hbm,
          lat_idx_v, lon_idx_v, lat_v, lon_v, sem1, sem2):
    wid = lax.axis_index("s") * _NC + lax.axis_index("c")
    base = wid * _BPW
    pltpu.sync_copy(lat_idx_hbm.at[pl.ds(base, _BPW)], lat_idx_v)
    pltpu.sync_copy(lon_idx_hbm.at[pl.ds(base, _BPW)], lon_idx_v)
    for c in range(_NCHUNK):
        cp1 = pltpu.async_copy(
            lat_t_hbm.at[lat_idx_v.at[pl.ds(c * _CH, _CH)]], lat_v, sem1)
        cp2 = pltpu.async_copy(
            lon_t_hbm.at[lon_idx_v.at[pl.ds(c * _CH, _CH)]], lon_v, sem2)
        cp1.wait()
        pltpu.sync_copy(
            lat_v, out_hbm.at[pl.ds(base + c * _CH, _CH), pl.ds(0, EMBED_DIM)])
        cp2.wait()
        pltpu.sync_copy(
            lon_v,
            out_hbm.at[pl.ds(base + c * _CH, _CH), pl.ds(EMBED_DIM, EMBED_DIM)])


def kernel(latitudes, longitudes, lat_table, lon_table):
    mesh = plsc.VectorSubcoreMesh(core_axis_name="c", subcore_axis_name="s")
    k = functools.partial(
        pl.kernel,
        mesh=mesh,
        out_type=jax.ShapeDtypeStruct((BATCH, 2 * EMBED_DIM), jnp.float32),
        compiler_params=pltpu.CompilerParams(use_tc_tiling_on_sc=False),
        scratch_types=[
            pltpu.VMEM((_BPW,), jnp.int32),
            pltpu.VMEM((_BPW,), jnp.int32),
            pltpu.VMEM((_CH, EMBED_DIM), jnp.float32),
            pltpu.VMEM((_CH, EMBED_DIM), jnp.float32),
            pltpu.SemaphoreType.DMA,
            pltpu.SemaphoreType.DMA,
        ],
    )(_body)
    return k(latitudes, longitudes, lat_table, lon_table)
